# baseline (speedup 1.0000x reference)
import functools

import jax
import jax.numpy as jnp
from jax import lax
from jax.experimental import pallas as pl
from jax.experimental.pallas import tpu as pltpu
from jax.experimental.pallas import tpu_sc as plsc

_LIST_OF_N = [100000] * 8
_H = len(_LIST_OF_N)
_N = _LIST_OF_N[0]
_D = 64

_INFO = plsc.get_sparse_core_info()
_NC = _INFO.num_cores
_NS = _INFO.num_subcores
_NW = _NC * _NS
_LANES = _INFO.num_lanes

_TOTAL = 4 * 4096 * _H
_PER_W = _TOTAL // _NW
_C = 128
_CHUNKS = _PER_W // _C
_NBUF = 4


def _sc_body(ids_hbm, table_hbm, drain_hbm, out_hbm, idx_v, rows0, rows1,
             rows2, rows3, g0, g1, g2, g3):
  w = lax.axis_index("s") * _NC + lax.axis_index("c")
  base = w * _PER_W

  pltpu.sync_copy(ids_hbm.at[lax.div(w, _H), lax.rem(w, _H)], idx_v)

  off = jnp.broadcast_to((lax.rem(w, _H) * _N).astype(jnp.int32), (_LANES,))

  def add_body(c, carry):
    for k in range(_C // _LANES):
      sl = pl.ds(k * _LANES, _LANES)
      idx_v[c, sl] = idx_v[c, sl] + off
    return carry

  lax.fori_loop(0, _CHUNKS, add_body, 0)

  bufs = (rows0, rows1, rows2, rows3)
  sems = (g0, g1, g2, g3)

  def start(c, b):
    def group(g, carry):
      v = idx_v[c, pl.ds(g * _LANES, _LANES)]
      vg = lax.shift_right_logical(v, 3)
      vs = lax.bitwise_and(v, 7)
      for j in range(_LANES):
        i = g * _LANES + j
        pltpu.async_copy(table_hbm.at[vg[j], vs[j]],
                         bufs[b].at[lax.div(i, 8), lax.rem(i, 8)], sems[b])
      return carry

    lax.fori_loop(0, _C // _LANES, group, 0)

  def wait(b):
    pltpu.make_async_copy(drain_hbm, bufs[b], sems[b]).wait()

  def store(c, b):
    pltpu.sync_copy(bufs[b],
                    out_hbm.at[pl.ds((base + c * _C) // 8, _C // 8)])

  for b in range(_NBUF):
    start(b, b)

  def outer(i, carry):
    c0 = i * _NBUF
    for b in range(_NBUF):
      c = c0 + b
      wait(b)
      store(c, b)
      start(c + _NBUF, b)
    return carry

  lax.fori_loop(0, (_CHUNKS - _NBUF) // _NBUF, outer, 0)

  for b in range(_NBUF):
    c = _CHUNKS - _NBUF + b
    wait(b)
    store(c, b)


_sc_call = functools.partial(
    pl.kernel,
    out_type=jax.ShapeDtypeStruct((_TOTAL // 8, 8, _D), jnp.float32),
    mesh=plsc.VectorSubcoreMesh(core_axis_name="c", subcore_axis_name="s"),
    scratch_types=[
        pltpu.VMEM((_CHUNKS, _C), jnp.int32),
        pltpu.VMEM((_C // 8, 8, _D), jnp.float32),
        pltpu.VMEM((_C // 8, 8, _D), jnp.float32),
        pltpu.VMEM((_C // 8, 8, _D), jnp.float32),
        pltpu.VMEM((_C // 8, 8, _D), jnp.float32),
        pltpu.SemaphoreType.DMA,
        pltpu.SemaphoreType.DMA,
        pltpu.SemaphoreType.DMA,
        pltpu.SemaphoreType.DMA,
    ],
    compiler_params=pltpu.CompilerParams(use_tc_tiling_on_sc=True),
)(_sc_body)


@jax.jit
def kernel(input_ids, embedding_weight):
  b, s, h = input_ids.shape
  ids = input_ids.transpose(0, 2, 1).reshape(b, h, _CHUNKS, _C)
  table = embedding_weight.reshape(embedding_weight.shape[0] // 8, 8, _D)
  drain = jnp.zeros((_C // 8, 8, _D), jnp.float32)
  out = _sc_call(ids, table, drain)
  return out.reshape(b, h, s, _D).transpose(0, 2, 1, 3)


# speedup vs baseline: 1.0016x; 1.0016x over previous
import functools

import jax
import jax.numpy as jnp
from jax import lax
from jax.experimental import pallas as pl
from jax.experimental.pallas import tpu as pltpu
from jax.experimental.pallas import tpu_sc as plsc

_LIST_OF_N = [100000] * 8
_H = len(_LIST_OF_N)
_N = _LIST_OF_N[0]
_D = 64

_INFO = plsc.get_sparse_core_info()
_NC = _INFO.num_cores
_NS = _INFO.num_subcores
_NW = _NC * _NS
_LANES = _INFO.num_lanes

_TOTAL = 4 * 4096 * _H
_PER_W = _TOTAL // _NW
_C = 128
_CHUNKS = _PER_W // _C
_NBUF = 2


def _sc_body(ids_hbm, table_hbm, drain_hbm, out_hbm, idx_v, rows0, rows1,
             g0, g1):
  w = lax.axis_index("s") * _NC + lax.axis_index("c")
  base = w * _PER_W

  pltpu.sync_copy(ids_hbm.at[lax.div(w, _H), lax.rem(w, _H)], idx_v)

  off = jnp.broadcast_to((lax.rem(w, _H) * _N).astype(jnp.int32), (_LANES,))

  def add_body(c, carry):
    for k in range(_C // _LANES):
      sl = pl.ds(k * _LANES, _LANES)
      idx_v[c, sl] = idx_v[c, sl] + off
    return carry

  lax.fori_loop(0, _CHUNKS, add_body, 0)

  bufs = (rows0, rows1)
  sems = (g0, g1)

  def start(c, b):
    def group(g, carry):
      v = idx_v[c, pl.ds(g * _LANES, _LANES)]
      vg = lax.shift_right_logical(v, 3)
      vs = lax.bitwise_and(v, 7)
      for j in range(_LANES):
        i = g * _LANES + j
        pltpu.async_copy(table_hbm.at[vg[j], vs[j]],
                         bufs[b].at[lax.div(i, 8), lax.rem(i, 8)], sems[b])
      return carry

    lax.fori_loop(0, _C // _LANES, group, 0)

  def wait(b):
    pltpu.make_async_copy(drain_hbm, bufs[b], sems[b]).wait()

  def store(c, b):
    pltpu.sync_copy(bufs[b],
                    out_hbm.at[pl.ds((base + c * _C) // 8, _C // 8)])

  for b in range(_NBUF):
    start(b, b)

  def outer(i, carry):
    c0 = i * _NBUF
    for b in range(_NBUF):
      c = c0 + b
      wait(b)
      store(c, b)
      start(c + _NBUF, b)
    return carry

  lax.fori_loop(0, (_CHUNKS - _NBUF) // _NBUF, outer, 0)

  for b in range(_NBUF):
    c = _CHUNKS - _NBUF + b
    wait(b)
    store(c, b)


_sc_call = functools.partial(
    pl.kernel,
    out_type=jax.ShapeDtypeStruct((_TOTAL // 8, 8, _D), jnp.float32),
    mesh=plsc.VectorSubcoreMesh(core_axis_name="c", subcore_axis_name="s"),
    scratch_types=[
        pltpu.VMEM((_CHUNKS, _C), jnp.int32),
        pltpu.VMEM((_C // 8, 8, _D), jnp.float32),
        pltpu.VMEM((_C // 8, 8, _D), jnp.float32),
        pltpu.SemaphoreType.DMA,
        pltpu.SemaphoreType.DMA,
    ],
    compiler_params=pltpu.CompilerParams(use_tc_tiling_on_sc=True),
)(_sc_body)


@jax.jit
def kernel(input_ids, embedding_weight):
  b, s, h = input_ids.shape
  ids = input_ids.transpose(0, 2, 1).reshape(b, h, _CHUNKS, _C)
  table = embedding_weight.reshape(embedding_weight.shape[0] // 8, 8, _D)
  drain = jnp.zeros((_C // 8, 8, _D), jnp.float32)
  out = _sc_call(ids, table, drain)
  return out.reshape(b, h, s, _D).transpose(0, 2, 1, 3)
